# baseline (speedup 1.0000x reference)
import functools

import jax
import jax.numpy as jnp
from jax import lax
from jax.experimental import pallas as pl
from jax.experimental.pallas import tpu as pltpu
from jax.experimental.pallas import tpu_sc as plsc

NC = 2
NS = 16
NW = NC * NS
B = 125


def _mesh():
    return plsc.VectorSubcoreMesh(
        core_axis_name="c", subcore_axis_name="s", num_cores=NC, num_subcores=NS
    )


_SC_PARAMS = pltpu.CompilerParams(use_tc_tiling_on_sc=False,
                                  needs_layout_passes=False)


def _make_deg(np_, n_chunk):
    rpt = np_ // NS

    @functools.partial(
        pl.kernel,
        out_type=jax.ShapeDtypeStruct((8, np_), jnp.float32),
        mesh=_mesh(),
        compiler_params=_SC_PARAMS,
        scratch_types=[
            pltpu.VMEM((n_chunk, B), jnp.int32),
            pltpu.VMEM((128,), jnp.float32),
            pltpu.VMEM((rpt,), jnp.float32),
            pltpu.VMEM_SHARED((np_,), jnp.float32),
        ],
    )
    def deg_kernel(dst_hbm, out_hbm, idx_v, ones_v, buf_v, acc_sh):
        c = lax.axis_index("c")
        s = lax.axis_index("s")
        wid = c * NS + s

        def fill_ones(i, cr):
            ones_v[pl.ds(i * 16, 16)] = jnp.ones((16,), jnp.float32)
            return cr

        lax.fori_loop(0, 128 // 16, fill_ones, 0)

        def fill_zero(i, cr):
            buf_v[pl.ds(i * 16, 16)] = jnp.zeros((16,), jnp.float32)
            return cr

        lax.fori_loop(0, rpt // 16, fill_zero, 0)
        pltpu.sync_copy(buf_v, acc_sh.at[pl.ds(s * rpt, rpt)])
        plsc.subcore_barrier()

        pltpu.sync_copy(dst_hbm.at[pl.ds(wid * n_chunk, n_chunk)], idx_v)

        def chunk(j, cr):
            pltpu.sync_copy(ones_v.at[pl.ds(0, B)], acc_sh.at[idx_v.at[j]], add=True)
            return cr

        lax.fori_loop(0, n_chunk, chunk, 0)
        plsc.subcore_barrier()
        pltpu.sync_copy(acc_sh.at[pl.ds(s * rpt, rpt)],
                        out_hbm.at[c, pl.ds(s * rpt, rpt)])

    return deg_kernel


def _make_agg(np_, d, n_chunk):
    rpt = np_ // NS

    @functools.partial(
        pl.kernel,
        out_type=jax.ShapeDtypeStruct((NC, d, np_), jnp.float32),
        mesh=_mesh(),
        compiler_params=_SC_PARAMS,
        scratch_types=[
            pltpu.VMEM((n_chunk, B), jnp.int32),
            pltpu.VMEM((n_chunk, B), jnp.int32),
            pltpu.VMEM((B, d), jnp.float32),
            pltpu.VMEM((B, d), jnp.float32),
            pltpu.VMEM((d, rpt), jnp.float32),
            pltpu.VMEM((rpt, d), jnp.float32),
            pltpu.SemaphoreType.DMA,
            pltpu.SemaphoreType.DMA,
            pltpu.VMEM_SHARED((np_, d), jnp.float32),
            pltpu.HBM((np_, d), jnp.float32),
            pltpu.HBM((np_, d), jnp.float32),
        ],
    )
    def agg_kernel(src_hbm, dst_hbm, gt_hbm, out_hbm, sidx_v, didx_v,
                   rows0_v, rows1_v, bufT_v, bufA_v, sem0, sem1, acc_sh,
                   gscr0, gscr1):
        c = lax.axis_index("c")
        s = lax.axis_index("s")
        wid = c * NS + s
        rows16 = lax.iota(jnp.int32, 16)

        pltpu.sync_copy(gt_hbm.at[:, pl.ds(s * rpt, rpt)], bufT_v)

        def col(i0, cr):
            base = jnp.full((16,), i0 * 16, jnp.int32)
            for k in range(16):
                for p in range(d // 16):
                    v = plsc.load_gather(bufT_v, [rows16 + 16 * p, base + k])
                    bufA_v[i0 * 16 + k, pl.ds(16 * p, 16)] = v
            return cr

        lax.fori_loop(0, rpt // 16, col, 0)

        @pl.when(c == 0)
        def _():
            pltpu.sync_copy(bufA_v, gscr0.at[pl.ds(s * rpt, rpt)])

        @pl.when(c == 1)
        def _():
            pltpu.sync_copy(bufA_v, gscr1.at[pl.ds(s * rpt, rpt)])

        zeros16 = jnp.zeros((16,), jnp.float32)

        def fill_zero(i0, cr):
            for k in range(16):
                for p in range(d // 16):
                    bufA_v[i0 * 16 + k, pl.ds(p * 16, 16)] = zeros16
            return cr

        lax.fori_loop(0, rpt // 16, fill_zero, 0)
        pltpu.sync_copy(bufA_v, acc_sh.at[pl.ds(s * rpt, rpt)])
        plsc.subcore_barrier()

        pltpu.sync_copy(src_hbm.at[pl.ds(wid * n_chunk, n_chunk)], sidx_v)
        pltpu.sync_copy(dst_hbm.at[pl.ds(wid * n_chunk, n_chunk)], didx_v)

        n_pairs = n_chunk // 2

        def run_pipe(gsrc):
            pltpu.async_copy(gsrc.at[sidx_v.at[0]], rows0_v, sem0)

            def pair(p, cr):
                j0 = 2 * p
                pltpu.async_copy(gsrc.at[sidx_v.at[j0 + 1]], rows1_v, sem1)
                pltpu.make_async_copy(gsrc.at[sidx_v.at[j0]], rows0_v, sem0).wait()
                pltpu.sync_copy(rows0_v, acc_sh.at[didx_v.at[j0]], add=True)

                @pl.when(p + 1 < n_pairs)
                def _():
                    pltpu.async_copy(gsrc.at[sidx_v.at[j0 + 2]], rows0_v, sem0)

                pltpu.make_async_copy(gsrc.at[sidx_v.at[j0 + 1]], rows1_v, sem1).wait()
                pltpu.sync_copy(rows1_v, acc_sh.at[didx_v.at[j0 + 1]], add=True)
                return cr

            lax.fori_loop(0, n_pairs, pair, 0)

        @pl.when(c == 0)
        def _():
            run_pipe(gscr0)

        @pl.when(c == 1)
        def _():
            run_pipe(gscr1)

        plsc.subcore_barrier()

        pltpu.sync_copy(acc_sh.at[pl.ds(s * rpt, rpt)], bufA_v)

        def col2(i0, cr):
            base = jnp.full((16,), i0 * 16, jnp.int32)
            for k in range(16):
                for p in range(d // 16):
                    plsc.store_scatter(
                        bufT_v, [rows16 + 16 * p, base + k],
                        bufA_v[i0 * 16 + k, pl.ds(16 * p, 16)])
            return cr

        lax.fori_loop(0, rpt // 16, col2, 0)
        pltpu.sync_copy(bufT_v, out_hbm.at[c, :, pl.ds(s * rpt, rpt)])

    return agg_kernel


def _make_tc1(np_, n, fin, h):
    def body(x, w1, degt, g1t):
        d = degt[...]
        dinv = lax.rsqrt(d[0:1] + d[1:2] + 1.0)
        hmt = lax.dot_general(w1[...], x[...], (((0,), (1,)), ((), ())),
                              preferred_element_type=jnp.float32)
        g1t[:, pl.ds(0, n)] = hmt * dinv[:, :n]

    return pl.pallas_call(
        body,
        out_shape=jax.ShapeDtypeStruct((h, np_), jnp.float32),
    )


def _make_tc2(np_, n, h, dp):
    def body(degt, p1t, g1t, w2, b1c, g2t):
        d = degt[...]
        dinv = lax.rsqrt(d[0:1] + d[1:2] + 1.0)
        p = p1t[...]
        ot = dinv * (p[0] + p[1] + g1t[...]) + b1c[...]
        ot = jnp.maximum(ot, 0.0)
        g2 = lax.dot_general(w2[...], ot, (((0,), (0,)), ((), ())),
                             preferred_element_type=jnp.float32)
        g2t[...] = g2 * dinv

    return pl.pallas_call(
        body,
        out_shape=jax.ShapeDtypeStruct((dp, np_), jnp.float32),
    )


def _make_tc3(np_, n, dp, cout):
    def body(degt, p2t, g2t, b2c, out):
        d = degt[...]
        dinv = lax.rsqrt(d[0:1] + d[1:2] + 1.0)
        p = p2t[...]
        z = dinv * (p[0] + p[1] + g2t[...]) + b2c[...]
        zs = z[:cout, :n]
        m = jnp.max(zs, axis=0, keepdims=True)
        e = jnp.exp(zs - m)
        lse = m + jnp.log(jnp.sum(e, axis=0, keepdims=True))
        out[...] = (zs - lse).T

    return pl.pallas_call(
        body,
        out_shape=jax.ShapeDtypeStruct((n, cout), jnp.float32),
    )


def kernel(x, edge_index, W1, b1, W2, b2):
    n, fin = x.shape
    e = edge_index.shape[1]
    h = W1.shape[1]
    cout = W2.shape[1]
    dp = 48
    np_ = ((n + 255) // 256) * 256
    assert e % (NW * B) == 0
    n_chunk = e // (NW * B)

    src = edge_index[0].astype(jnp.int32).reshape(NW * n_chunk, B)
    dst = edge_index[1].astype(jnp.int32).reshape(NW * n_chunk, B)
    w2p = jnp.zeros((h, dp), jnp.float32).at[:, :cout].set(W2)
    b1c = b1[:, None]
    b2c = jnp.zeros((dp, 1), jnp.float32).at[:cout, 0].set(b2)

    degt = _make_deg(np_, n_chunk)(dst)
    g1t = _make_tc1(np_, n, fin, h)(x, W1, degt)
    p1t = _make_agg(np_, h, n_chunk)(src, dst, g1t)
    g2t = _make_tc2(np_, n, h, dp)(degt, p1t, g1t, w2p, b1c)
    p2t = _make_agg(np_, dp, n_chunk)(src, dst, g2t)
    out = _make_tc3(np_, n, dp, cout)(degt, p2t, g2t, b2c)
    return out


# speedup vs baseline: 1.0614x; 1.0614x over previous
import functools

import jax
import jax.numpy as jnp
from jax import lax
from jax.experimental import pallas as pl
from jax.experimental.pallas import tpu as pltpu
from jax.experimental.pallas import tpu_sc as plsc

NC = 2
NS = 16
NW = NC * NS
B = 125


def _mesh():
    return plsc.VectorSubcoreMesh(
        core_axis_name="c", subcore_axis_name="s", num_cores=NC, num_subcores=NS
    )


_SC_PARAMS = pltpu.CompilerParams(use_tc_tiling_on_sc=False,
                                  needs_layout_passes=False)


def _make_deg(np_, er, mrows):
    rpt = np_ // NS

    @functools.partial(
        pl.kernel,
        out_type=jax.ShapeDtypeStruct((8, np_), jnp.float32),
        mesh=_mesh(),
        compiler_params=_SC_PARAMS,
        scratch_types=[
            pltpu.VMEM((mrows, 128), jnp.int32),
            pltpu.VMEM((128,), jnp.float32),
            pltpu.VMEM((rpt,), jnp.float32),
            pltpu.VMEM_SHARED((np_,), jnp.float32),
        ],
    )
    def deg_kernel(ei_hbm, out_hbm, idx_v, ones_v, buf_v, acc_sh):
        c = lax.axis_index("c")
        s = lax.axis_index("s")
        wid = c * NS + s
        r0 = (er * wid) // NW
        nr = (er * (wid + 1)) // NW - r0

        def fill_ones(i, cr):
            ones_v[pl.ds(i * 16, 16)] = jnp.ones((16,), jnp.float32)
            return cr

        lax.fori_loop(0, 128 // 16, fill_ones, 0)

        def fill_zero(i, cr):
            buf_v[pl.ds(i * 16, 16)] = jnp.zeros((16,), jnp.float32)
            return cr

        lax.fori_loop(0, rpt // 16, fill_zero, 0)
        pltpu.sync_copy(buf_v, acc_sh.at[pl.ds(s * rpt, rpt)])
        plsc.subcore_barrier()

        pltpu.sync_copy(ei_hbm.at[1, pl.ds(r0, mrows)], idx_v)

        def chunk(j, cr):
            pltpu.sync_copy(ones_v, acc_sh.at[idx_v.at[j]], add=True)
            return cr

        lax.fori_loop(0, nr, chunk, 0)
        plsc.subcore_barrier()
        pltpu.sync_copy(acc_sh.at[pl.ds(s * rpt, rpt)],
                        out_hbm.at[c, pl.ds(s * rpt, rpt)])

    return deg_kernel


def _make_agg(np_, d, er, mrows):
    rpt = np_ // NS

    @functools.partial(
        pl.kernel,
        out_type=jax.ShapeDtypeStruct((NC, d, np_), jnp.float32),
        mesh=_mesh(),
        compiler_params=_SC_PARAMS,
        scratch_types=[
            pltpu.VMEM((mrows, 128), jnp.int32),
            pltpu.VMEM((mrows, 128), jnp.int32),
            pltpu.VMEM((128, d), jnp.float32),
            pltpu.VMEM((128, d), jnp.float32),
            pltpu.VMEM((d, rpt), jnp.float32),
            pltpu.VMEM((rpt, d), jnp.float32),
            pltpu.SemaphoreType.DMA,
            pltpu.SemaphoreType.DMA,
            pltpu.VMEM_SHARED((np_, d), jnp.float32),
            pltpu.HBM((np_, d), jnp.float32),
            pltpu.HBM((np_, d), jnp.float32),
        ],
    )
    def agg_kernel(ei_hbm, gt_hbm, out_hbm, sidx_v, didx_v,
                   rows0_v, rows1_v, bufT_v, bufA_v, sem0, sem1, acc_sh,
                   gscr0, gscr1):
        c = lax.axis_index("c")
        s = lax.axis_index("s")
        wid = c * NS + s
        r0 = (er * wid) // NW
        nr = (er * (wid + 1)) // NW - r0
        rows16 = lax.iota(jnp.int32, 16)

        pltpu.sync_copy(gt_hbm.at[:, pl.ds(s * rpt, rpt)], bufT_v)

        def col(i0, cr):
            base = jnp.full((16,), i0 * 16, jnp.int32)
            for k in range(16):
                for p in range(d // 16):
                    v = plsc.load_gather(bufT_v, [rows16 + 16 * p, base + k])
                    bufA_v[i0 * 16 + k, pl.ds(16 * p, 16)] = v
            return cr

        lax.fori_loop(0, rpt // 16, col, 0)

        @pl.when(c == 0)
        def _():
            pltpu.sync_copy(bufA_v, gscr0.at[pl.ds(s * rpt, rpt)])

        @pl.when(c == 1)
        def _():
            pltpu.sync_copy(bufA_v, gscr1.at[pl.ds(s * rpt, rpt)])

        zeros16 = jnp.zeros((16,), jnp.float32)

        def fill_zero(i0, cr):
            for k in range(16):
                for p in range(d // 16):
                    bufA_v[i0 * 16 + k, pl.ds(p * 16, 16)] = zeros16
            return cr

        lax.fori_loop(0, rpt // 16, fill_zero, 0)
        pltpu.sync_copy(bufA_v, acc_sh.at[pl.ds(s * rpt, rpt)])
        plsc.subcore_barrier()

        pltpu.sync_copy(ei_hbm.at[0, pl.ds(r0, mrows)], sidx_v)
        pltpu.sync_copy(ei_hbm.at[1, pl.ds(r0, mrows)], didx_v)

        def run_pipe(gsrc):
            pltpu.async_copy(gsrc.at[sidx_v.at[0]], rows0_v, sem0)

            def pair(p, cr):
                j0 = 2 * p

                @pl.when(j0 + 1 < nr)
                def _():
                    pltpu.async_copy(gsrc.at[sidx_v.at[j0 + 1]], rows1_v, sem1)

                pltpu.make_async_copy(gsrc.at[sidx_v.at[j0]], rows0_v, sem0).wait()
                pltpu.sync_copy(rows0_v, acc_sh.at[didx_v.at[j0]], add=True)

                @pl.when(j0 + 2 < nr)
                def _():
                    pltpu.async_copy(gsrc.at[sidx_v.at[j0 + 2]], rows0_v, sem0)

                @pl.when(j0 + 1 < nr)
                def _():
                    pltpu.make_async_copy(gsrc.at[sidx_v.at[j0 + 1]], rows1_v,
                                          sem1).wait()
                    pltpu.sync_copy(rows1_v, acc_sh.at[didx_v.at[j0 + 1]], add=True)

                return cr

            lax.fori_loop(0, (nr + 1) // 2, pair, 0)

        @pl.when(c == 0)
        def _():
            run_pipe(gscr0)

        @pl.when(c == 1)
        def _():
            run_pipe(gscr1)

        plsc.subcore_barrier()

        pltpu.sync_copy(acc_sh.at[pl.ds(s * rpt, rpt)], bufA_v)

        def col2(i0, cr):
            base = jnp.full((16,), i0 * 16, jnp.int32)
            for k in range(16):
                for p in range(d // 16):
                    plsc.store_scatter(
                        bufT_v, [rows16 + 16 * p, base + k],
                        bufA_v[i0 * 16 + k, pl.ds(16 * p, 16)])
            return cr

        lax.fori_loop(0, rpt // 16, col2, 0)
        pltpu.sync_copy(bufT_v, out_hbm.at[c, :, pl.ds(s * rpt, rpt)])

    return agg_kernel


def _make_tc1(np_, n, fin, h):
    def body(x, w1, degt, g1t):
        d = degt[...]
        dinv = lax.rsqrt(d[0:1] + d[1:2] + 1.0)
        hmt = lax.dot_general(w1[...], x[...], (((0,), (1,)), ((), ())),
                              preferred_element_type=jnp.float32)
        g1t[:, pl.ds(0, n)] = hmt * dinv[:, :n]

    return pl.pallas_call(
        body,
        out_shape=jax.ShapeDtypeStruct((h, np_), jnp.float32),
    )


def _make_tc2(np_, n, h, dp):
    def body(degt, p1t, g1t, w2, b1c, g2t):
        d = degt[...]
        dinv = lax.rsqrt(d[0:1] + d[1:2] + 1.0)
        p = p1t[...]
        ot = dinv * (p[0] + p[1] + g1t[...]) + b1c[...]
        ot = jnp.maximum(ot, 0.0)
        g2 = lax.dot_general(w2[...], ot, (((0,), (0,)), ((), ())),
                             preferred_element_type=jnp.float32)
        g2t[...] = g2 * dinv

    return pl.pallas_call(
        body,
        out_shape=jax.ShapeDtypeStruct((dp, np_), jnp.float32),
    )


def _make_tc3(np_, n, dp, cout):
    def body(degt, p2t, g2t, b2c, out):
        d = degt[...]
        dinv = lax.rsqrt(d[0:1] + d[1:2] + 1.0)
        p = p2t[...]
        z = dinv * (p[0] + p[1] + g2t[...]) + b2c[...]
        zs = z[:cout, :n]
        m = jnp.max(zs, axis=0, keepdims=True)
        e = jnp.exp(zs - m)
        lse = m + jnp.log(jnp.sum(e, axis=0, keepdims=True))
        out[...] = (zs - lse).T

    return pl.pallas_call(
        body,
        out_shape=jax.ShapeDtypeStruct((n, cout), jnp.float32),
    )


def kernel(x, edge_index, W1, b1, W2, b2):
    n, fin = x.shape
    e = edge_index.shape[1]
    h = W1.shape[1]
    cout = W2.shape[1]
    dp = 48
    np_ = ((n + 255) // 256) * 256
    assert e % 128 == 0
    er = e // 128
    mrows = er - (er * (NW - 1)) // NW

    ei = edge_index.astype(jnp.int32).reshape(2, er, 128)
    w2p = jnp.zeros((h, dp), jnp.float32).at[:, :cout].set(W2)
    b1c = b1[:, None]
    b2c = jnp.zeros((dp, 1), jnp.float32).at[:cout, 0].set(b2)

    degt = _make_deg(np_, er, mrows)(ei)
    g1t = _make_tc1(np_, n, fin, h)(x, W1, degt)
    p1t = _make_agg(np_, h, er, mrows)(ei, g1t)
    g2t = _make_tc2(np_, n, h, dp)(degt, p1t, g1t, w2p, b1c)
    p2t = _make_agg(np_, dp, er, mrows)(ei, g2t)
    out = _make_tc3(np_, n, dp, cout)(degt, p2t, g2t, b2c)
    return out
